# baseline (speedup 1.0000x reference)
import functools

import jax
import jax.numpy as jnp
from jax.experimental import pallas as pl
from jax.experimental.pallas import tpu as pltpu

_BATCH_BLOCK = 16


def _scale_kernel(temp_ref, gid_ref, x_ref, o_ref):
    g = gid_ref[...]
    num_groups = temp_ref.shape[0]
    s = jnp.zeros(g.shape, dtype=jnp.float32)
    for gid in range(num_groups):
        s = jnp.where(g == gid, 1.0 / temp_ref[gid], s)
    o_ref[...] = x_ref[...] * s


def kernel(logits, group_ids, temperatures):
    batch, vocab = logits.shape
    bm = min(_BATCH_BLOCK, batch)
    bn = vocab
    grid = (pl.cdiv(batch, bm), pl.cdiv(vocab, bn))
    gid2 = group_ids.reshape(batch, 1)
    return pl.pallas_call(
        _scale_kernel,
        grid=grid,
        in_specs=[
            pl.BlockSpec(memory_space=pltpu.SMEM),
            pl.BlockSpec((bm, 1), lambda i, j: (i, 0)),
            pl.BlockSpec((bm, bn), lambda i, j: (i, j)),
        ],
        out_specs=pl.BlockSpec((bm, bn), lambda i, j: (i, j)),
        out_shape=jax.ShapeDtypeStruct((batch, vocab), logits.dtype),
    )(temperatures, gid2, logits)


# speedup vs baseline: 3.8157x; 3.8157x over previous
import jax
import jax.numpy as jnp
from jax.experimental import pallas as pl
from jax.experimental.pallas import tpu as pltpu

_VOCAB_BLOCK = 2048


def _scale_kernel(temp_ref, gid_ref, x_ref, o_ref):
    g = gid_ref[...]
    num_groups = temp_ref.shape[0]
    s = jnp.zeros(g.shape, dtype=jnp.float32)
    for gid in range(num_groups):
        s = jnp.where(g == gid, 1.0 / temp_ref[gid], s)
    o_ref[...] = x_ref[...] * s


def kernel(logits, group_ids, temperatures):
    batch, vocab = logits.shape
    bn = _VOCAB_BLOCK
    xt = logits.T
    gid2 = group_ids.reshape(1, batch)
    out_t = pl.pallas_call(
        _scale_kernel,
        grid=(pl.cdiv(vocab, bn),),
        in_specs=[
            pl.BlockSpec(memory_space=pltpu.SMEM),
            pl.BlockSpec((1, batch), lambda j: (0, 0)),
            pl.BlockSpec((bn, batch), lambda j: (j, 0)),
        ],
        out_specs=pl.BlockSpec((bn, batch), lambda j: (j, 0)),
        out_shape=jax.ShapeDtypeStruct((vocab, batch), logits.dtype),
    )(temperatures, gid2, xt)
    return out_t.T


# speedup vs baseline: 3.8318x; 1.0042x over previous
import jax
import jax.numpy as jnp
from jax.experimental import pallas as pl
from jax.experimental.pallas import tpu as pltpu

_VOCAB_BLOCK = 3072


def _scale_kernel(temp_ref, gid_ref, x_ref, o_ref):
    g = gid_ref[...]
    num_groups = temp_ref.shape[0]
    s = jnp.zeros(g.shape, dtype=jnp.float32)
    for gid in range(num_groups):
        s = jnp.where(g == gid, 1.0 / temp_ref[gid], s)
    o_ref[...] = x_ref[...] * s


def kernel(logits, group_ids, temperatures):
    batch, vocab = logits.shape
    bn = _VOCAB_BLOCK
    xt = logits.T
    gid2 = group_ids.reshape(1, batch)
    out_t = pl.pallas_call(
        _scale_kernel,
        grid=(pl.cdiv(vocab, bn),),
        in_specs=[
            pl.BlockSpec(memory_space=pltpu.SMEM),
            pl.BlockSpec((1, batch), lambda j: (0, 0)),
            pl.BlockSpec((bn, batch), lambda j: (j, 0)),
        ],
        out_specs=pl.BlockSpec((bn, batch), lambda j: (j, 0)),
        out_shape=jax.ShapeDtypeStruct((vocab, batch), logits.dtype),
    )(temperatures, gid2, xt)
    return out_t.T


# speedup vs baseline: 3.8372x; 1.0014x over previous
import jax
import jax.numpy as jnp
from jax.experimental import pallas as pl
from jax.experimental.pallas import tpu as pltpu

_VOCAB_BLOCK = 3584


def _scale_kernel(temp_ref, gid_ref, x_ref, o_ref):
    g = gid_ref[...]
    num_groups = temp_ref.shape[0]
    s = jnp.zeros(g.shape, dtype=jnp.float32)
    for gid in range(num_groups):
        s = jnp.where(g == gid, 1.0 / temp_ref[gid], s)
    o_ref[...] = x_ref[...] * s


def kernel(logits, group_ids, temperatures):
    batch, vocab = logits.shape
    bn = _VOCAB_BLOCK
    xt = logits.T
    gid2 = group_ids.reshape(1, batch)
    out_t = pl.pallas_call(
        _scale_kernel,
        grid=(pl.cdiv(vocab, bn),),
        in_specs=[
            pl.BlockSpec(memory_space=pltpu.SMEM),
            pl.BlockSpec((1, batch), lambda j: (0, 0)),
            pl.BlockSpec((bn, batch), lambda j: (j, 0)),
        ],
        out_specs=pl.BlockSpec((bn, batch), lambda j: (j, 0)),
        out_shape=jax.ShapeDtypeStruct((vocab, batch), logits.dtype),
    )(temperatures, gid2, xt)
    return out_t.T
